# Initial kernel scaffold; baseline (speedup 1.0000x reference)
import jax
import jax.numpy as jnp
from jax.experimental import pallas as pl


def kernel(hidden_states, gate_weight):
    raise NotImplementedError("write your pallas kernel here")


# speedup vs baseline: 1.0073x; 1.0073x over previous
import functools

import jax
import jax.numpy as jnp
from jax.experimental import pallas as pl
from jax.experimental.pallas import tpu as pltpu

HIDDEN = 4096
EXPERTS = 64
K = 8
BLK = 512


def _router_body(x_ref, w_ref, rw_ref, idx_ref):
    logits = jnp.dot(x_ref[...], w_ref[...], preferred_element_type=jnp.float32)
    iota = jax.lax.broadcasted_iota(jnp.int32, (BLK, EXPERTS), 1)

    vals = logits
    top_vals = []
    top_idx = []
    for _ in range(K):
        m = jnp.max(vals, axis=-1, keepdims=True)
        is_max = vals == m
        idx = jnp.min(jnp.where(is_max, iota, EXPERTS), axis=-1, keepdims=True)
        top_vals.append(m)
        top_idx.append(idx)
        vals = jnp.where(iota == idx, -jnp.inf, vals)

    tv = jnp.concatenate(top_vals, axis=1)
    ti = jnp.concatenate(top_idx, axis=1)
    e = jnp.exp(tv - tv[:, :1])
    rw_ref[...] = e / jnp.sum(e, axis=-1, keepdims=True)
    idx_ref[...] = ti


@jax.jit
def kernel(hidden_states, gate_weight):
    b, s, d = hidden_states.shape
    n_tok = b * s
    x2d = hidden_states.reshape(n_tok, d)
    wt = gate_weight.T

    grid = (n_tok // BLK,)
    rw, idx = pl.pallas_call(
        _router_body,
        grid=grid,
        in_specs=[
            pl.BlockSpec((BLK, d), lambda i: (i, 0)),
            pl.BlockSpec((d, EXPERTS), lambda i: (0, 0)),
        ],
        out_specs=[
            pl.BlockSpec((BLK, K), lambda i: (i, 0)),
            pl.BlockSpec((BLK, K), lambda i: (i, 0)),
        ],
        out_shape=[
            jax.ShapeDtypeStruct((n_tok, K), jnp.float32),
            jax.ShapeDtypeStruct((n_tok, K), jnp.int32),
        ],
        compiler_params=pltpu.CompilerParams(
            dimension_semantics=("arbitrary",),
        ),
    )(x2d, wt)

    return rw.reshape(b, s, K), idx.reshape(b, s, K)


# speedup vs baseline: 1.1603x; 1.1519x over previous
import functools

import jax
import jax.numpy as jnp
from jax.experimental import pallas as pl
from jax.experimental.pallas import tpu as pltpu

HIDDEN = 4096
EXPERTS = 64
K = 8
BLK = 512


def _router_body(x_ref, w_ref, rw_ref, idx_ref):
    logits = jnp.dot(x_ref[...], w_ref[...], preferred_element_type=jnp.float32)
    riota = (
        jnp.int32(EXPERTS - 1)
        - jax.lax.broadcasted_iota(jnp.int32, (BLK, EXPERTS), 1)
    ).astype(jnp.float32)

    vals = logits
    top_vals = []
    top_ridx = []
    for _ in range(K):
        m = jnp.max(vals, axis=-1, keepdims=True)
        masked_iota = jnp.where(vals == m, riota, -1.0)
        r = jnp.max(masked_iota, axis=-1, keepdims=True)
        top_vals.append(m)
        top_ridx.append(r)
        vals = jnp.where(masked_iota == r, -jnp.inf, vals)

    tv = jnp.concatenate(top_vals, axis=1)
    tr = jnp.concatenate(top_ridx, axis=1)
    e = jnp.exp(tv - tv[:, :1])
    rw_ref[...] = e / jnp.sum(e, axis=-1, keepdims=True)
    idx_ref[...] = (jnp.float32(EXPERTS - 1) - tr).astype(jnp.int32)


@jax.jit
def kernel(hidden_states, gate_weight):
    b, s, d = hidden_states.shape
    n_tok = b * s
    x2d = hidden_states.reshape(n_tok, d)
    wt = gate_weight.T

    grid = (n_tok // BLK,)
    rw, idx = pl.pallas_call(
        _router_body,
        grid=grid,
        in_specs=[
            pl.BlockSpec((BLK, d), lambda i: (i, 0)),
            pl.BlockSpec((d, EXPERTS), lambda i: (0, 0)),
        ],
        out_specs=[
            pl.BlockSpec((BLK, K), lambda i: (i, 0)),
            pl.BlockSpec((BLK, K), lambda i: (i, 0)),
        ],
        out_shape=[
            jax.ShapeDtypeStruct((n_tok, K), jnp.float32),
            jax.ShapeDtypeStruct((n_tok, K), jnp.int32),
        ],
        compiler_params=pltpu.CompilerParams(
            dimension_semantics=("arbitrary",),
        ),
    )(x2d, wt)

    return rw.reshape(b, s, K), idx.reshape(b, s, K)


# speedup vs baseline: 1.2690x; 1.0937x over previous
import functools

import jax
import jax.numpy as jnp
from jax.experimental import pallas as pl
from jax.experimental.pallas import tpu as pltpu

HIDDEN = 4096
EXPERTS = 64
K = 8
BLK = 1024


def _router_body(x_ref, w_ref, rw_ref, idx_ref):
    logits = jnp.dot(x_ref[...], w_ref[...], preferred_element_type=jnp.float32)
    riota = (
        jnp.int32(EXPERTS - 1)
        - jax.lax.broadcasted_iota(jnp.int32, (BLK, EXPERTS), 1)
    ).astype(jnp.float32)

    vals = logits
    top_vals = []
    top_ridx = []
    for _ in range(K):
        m = jnp.max(vals, axis=-1, keepdims=True)
        masked_iota = jnp.where(vals == m, riota, -1.0)
        r = jnp.max(masked_iota, axis=-1, keepdims=True)
        top_vals.append(m)
        top_ridx.append(r)
        vals = jnp.where(masked_iota == r, -jnp.inf, vals)

    tv = jnp.concatenate(top_vals, axis=1)
    tr = jnp.concatenate(top_ridx, axis=1)
    e = jnp.exp(tv - tv[:, :1])
    rw_ref[...] = e / jnp.sum(e, axis=-1, keepdims=True)
    idx_ref[...] = (jnp.float32(EXPERTS - 1) - tr).astype(jnp.int32)


@jax.jit
def kernel(hidden_states, gate_weight):
    b, s, d = hidden_states.shape
    n_tok = b * s
    x2d = hidden_states.reshape(n_tok, d)
    wt = gate_weight.T

    grid = (n_tok // BLK,)
    rw, idx = pl.pallas_call(
        _router_body,
        grid=grid,
        in_specs=[
            pl.BlockSpec((BLK, d), lambda i: (i, 0)),
            pl.BlockSpec((d, EXPERTS), lambda i: (0, 0)),
        ],
        out_specs=[
            pl.BlockSpec((BLK, K), lambda i: (i, 0)),
            pl.BlockSpec((BLK, K), lambda i: (i, 0)),
        ],
        out_shape=[
            jax.ShapeDtypeStruct((n_tok, K), jnp.float32),
            jax.ShapeDtypeStruct((n_tok, K), jnp.int32),
        ],
        compiler_params=pltpu.CompilerParams(
            dimension_semantics=("parallel",),
        ),
    )(x2d, wt)

    return rw.reshape(b, s, K), idx.reshape(b, s, K)


# speedup vs baseline: 1.4254x; 1.1232x over previous
import functools

import jax
import jax.numpy as jnp
from jax.experimental import pallas as pl
from jax.experimental.pallas import tpu as pltpu

HIDDEN = 4096
EXPERTS = 64
K = 8
BLK = 1024
CHUNK = 256


def _router_body(x_ref, w_ref, rw_ref, idx_ref):
    w = w_ref[...]
    riota = (
        jnp.int32(EXPERTS - 1)
        - jax.lax.broadcasted_iota(jnp.int32, (CHUNK, EXPERTS), 1)
    ).astype(jnp.float32)

    for c in range(BLK // CHUNK):
        sl = pl.ds(c * CHUNK, CHUNK)
        logits = jnp.dot(x_ref[sl, :], w, preferred_element_type=jnp.float32)

        vals = logits
        top_vals = []
        top_ridx = []
        for _ in range(K):
            m = jnp.max(vals, axis=-1, keepdims=True)
            masked_iota = jnp.where(vals == m, riota, -1.0)
            r = jnp.max(masked_iota, axis=-1, keepdims=True)
            top_vals.append(m)
            top_ridx.append(r)
            vals = jnp.where(riota == r, -jnp.inf, vals)

        tv = jnp.concatenate(top_vals, axis=1)
        tr = jnp.concatenate(top_ridx, axis=1)
        e = jnp.exp(tv - tv[:, :1])
        rw_ref[sl, :] = e / jnp.sum(e, axis=-1, keepdims=True)
        idx_ref[sl, :] = (jnp.float32(EXPERTS - 1) - tr).astype(jnp.int32)


@jax.jit
def kernel(hidden_states, gate_weight):
    b, s, d = hidden_states.shape
    n_tok = b * s
    x2d = hidden_states.reshape(n_tok, d)
    wt = gate_weight.T

    grid = (n_tok // BLK,)
    rw, idx = pl.pallas_call(
        _router_body,
        grid=grid,
        in_specs=[
            pl.BlockSpec((BLK, d), lambda i: (i, 0)),
            pl.BlockSpec((d, EXPERTS), lambda i: (0, 0)),
        ],
        out_specs=[
            pl.BlockSpec((BLK, K), lambda i: (i, 0)),
            pl.BlockSpec((BLK, K), lambda i: (i, 0)),
        ],
        out_shape=[
            jax.ShapeDtypeStruct((n_tok, K), jnp.float32),
            jax.ShapeDtypeStruct((n_tok, K), jnp.int32),
        ],
        compiler_params=pltpu.CompilerParams(
            dimension_semantics=("parallel",),
        ),
    )(x2d, wt)

    return rw.reshape(b, s, K), idx.reshape(b, s, K)
